# baseline (speedup 1.0000x reference)
import functools

import jax
import jax.numpy as jnp
from jax import lax
from jax.experimental import pallas as pl
from jax.experimental.pallas import tpu as pltpu
from jax.experimental.pallas import tpu_sc as plsc

VOCAB = 1000000
D = 64
B = 16384 * 50
NC, NS = 2, 16
NW = NC * NS
BPW = B // NW
CHUNK = 512
NCHUNK = BPW // CHUNK
NBUF = 2

_mesh = plsc.VectorSubcoreMesh(core_axis_name="c", subcore_axis_name="s")


@functools.partial(
    pl.kernel,
    mesh=_mesh,
    out_type=jax.ShapeDtypeStruct((B, D), jnp.float32),
    compiler_params=pltpu.CompilerParams(use_tc_tiling_on_sc=False),
    scratch_types=[
        pltpu.VMEM((BPW,), jnp.int32),
        pltpu.VMEM((NBUF, CHUNK, D), jnp.float32),
        pltpu.SemaphoreType.DMA,
        pltpu.SemaphoreType.DMA,
    ],
)
def _gather_kernel(idx_hbm, table_hbm, out_hbm, idx_v, rows_v, sem_g, sem_w):
    wid = lax.axis_index("s") * NC + lax.axis_index("c")
    base = wid * BPW

    pltpu.sync_copy(idx_hbm.at[pl.ds(base, BPW)], idx_v)

    def launch_gather(cc, buf):
        pltpu.async_copy(
            table_hbm.at[idx_v.at[pl.ds(cc * CHUNK, CHUNK)]],
            rows_v.at[buf],
            sem_g,
        )

    def launch_write(cc, buf):
        pltpu.async_copy(
            rows_v.at[buf],
            out_hbm.at[pl.ds(base + cc * CHUNK, CHUNK)],
            sem_w,
        )

    def wait_gather(buf):
        pltpu.make_async_copy(
            table_hbm.at[pl.ds(0, CHUNK)], rows_v.at[buf], sem_g
        ).wait()

    def wait_write(buf):
        pltpu.make_async_copy(
            rows_v.at[buf], out_hbm.at[pl.ds(0, CHUNK)], sem_w
        ).wait()

    for j in range(NBUF - 1):
        launch_gather(j, j)

    @pl.loop(0, NCHUNK, step=NBUF)
    def _chunks(c0):
        for b in range(NBUF):
            c = c0 + b
            bn = (b - 1) % NBUF

            @pl.when(c >= 1)
            def _():
                wait_write(bn)

            @pl.when(c + NBUF - 1 < NCHUNK)
            def _():
                launch_gather(c + NBUF - 1, bn)

            wait_gather(b)
            launch_write(c, b)

    wait_write((NCHUNK - 1) % NBUF)


def kernel(inputs, weight):
    idx = inputs.reshape(-1).astype(jnp.int32)
    out = _gather_kernel(idx, weight)
    return out.reshape(inputs.shape + (D,))


# speedup vs baseline: 1.0017x; 1.0017x over previous
import functools

import jax
import jax.numpy as jnp
from jax import lax
from jax.experimental import pallas as pl
from jax.experimental.pallas import tpu as pltpu
from jax.experimental.pallas import tpu_sc as plsc

VOCAB = 1000000
D = 64
B = 16384 * 50
NC, NS = 2, 16
NW = NC * NS
BPW = B // NW
CHUNK = 256
NCHUNK = BPW // CHUNK
NBUF = 4

_mesh = plsc.VectorSubcoreMesh(core_axis_name="c", subcore_axis_name="s")


@functools.partial(
    pl.kernel,
    mesh=_mesh,
    out_type=jax.ShapeDtypeStruct((B, D), jnp.float32),
    compiler_params=pltpu.CompilerParams(use_tc_tiling_on_sc=False),
    scratch_types=[
        pltpu.VMEM((BPW,), jnp.int32),
        pltpu.VMEM((NBUF, CHUNK, D), jnp.float32),
        pltpu.SemaphoreType.DMA,
        pltpu.SemaphoreType.DMA,
    ],
)
def _gather_kernel(idx_hbm, table_hbm, out_hbm, idx_v, rows_v, sem_g, sem_w):
    wid = lax.axis_index("s") * NC + lax.axis_index("c")
    base = wid * BPW

    pltpu.sync_copy(idx_hbm.at[pl.ds(base, BPW)], idx_v)

    def launch_gather(cc, buf):
        pltpu.async_copy(
            table_hbm.at[idx_v.at[pl.ds(cc * CHUNK, CHUNK)]],
            rows_v.at[buf],
            sem_g,
        )

    def launch_write(cc, buf):
        pltpu.async_copy(
            rows_v.at[buf],
            out_hbm.at[pl.ds(base + cc * CHUNK, CHUNK)],
            sem_w,
        )

    def wait_gather(buf):
        pltpu.make_async_copy(
            table_hbm.at[pl.ds(0, CHUNK)], rows_v.at[buf], sem_g
        ).wait()

    def wait_write(buf):
        pltpu.make_async_copy(
            rows_v.at[buf], out_hbm.at[pl.ds(0, CHUNK)], sem_w
        ).wait()

    for j in range(NBUF - 1):
        launch_gather(j, j)

    @pl.loop(0, NCHUNK, step=NBUF)
    def _chunks(c0):
        for b in range(NBUF):
            c = c0 + b
            bn = (b - 1) % NBUF

            @pl.when(c >= 1)
            def _():
                wait_write(bn)

            @pl.when(c + NBUF - 1 < NCHUNK)
            def _():
                launch_gather(c + NBUF - 1, bn)

            wait_gather(b)
            launch_write(c, b)

    wait_write((NCHUNK - 1) % NBUF)


def kernel(inputs, weight):
    idx = inputs.reshape(-1).astype(jnp.int32)
    out = _gather_kernel(idx, weight)
    return out.reshape(inputs.shape + (D,))
